# Initial kernel scaffold; baseline (speedup 1.0000x reference)
import jax
import jax.numpy as jnp
from jax.experimental import pallas as pl


def kernel(x, edge_index, edge_weight, W0, W1, b):
    raise NotImplementedError("write your pallas kernel here")


# speedup vs baseline: 1.9192x; 1.9192x over previous
import jax
import jax.numpy as jnp
from jax import lax
from jax.experimental import pallas as pl
from jax.experimental.pallas import tpu as pltpu
from jax.experimental.pallas import tpu_sc as plsc

N_NODES = 10000
N_EDGES = 320000
D = 128
DH = D // 2
NC = 2
NS = 16
LANES = 16

EDGES_PER_TILE = N_EDGES // NS
CHUNK = 80
N_CHUNKS = EDGES_PER_TILE // CHUNK
ROWS_PER_TILE = 632
N_PAD = ROWS_PER_TILE * NS


def _spmm_body(h_hbm, src_hbm, dst_hbm, w_hbm, out_hbm,
               idx_src_v, idx_dst_v, rows_v, zbuf_v, w_v, acc_shared, sem):
    core = lax.axis_index("c")
    sub = lax.axis_index("s")

    zv = jnp.zeros((LANES,), jnp.float32)

    def _zero(j, _):
        r = j // (DH // LANES)
        k = j % (DH // LANES)
        zbuf_v[r, pl.ds(k * LANES, LANES)] = zv
        return ()

    lax.fori_loop(0, ROWS_PER_TILE * (DH // LANES), _zero, (), unroll=8)
    pltpu.sync_copy(zbuf_v, acc_shared.at[pl.ds(sub * ROWS_PER_TILE, ROWS_PER_TILE)])
    plsc.subcore_barrier()

    def _edge_chunk(i, _):
        base = sub * EDGES_PER_TILE + i * CHUNK
        pltpu.sync_copy(src_hbm.at[pl.ds(base, CHUNK)], idx_src_v)
        pltpu.sync_copy(dst_hbm.at[pl.ds(base, CHUNK)], idx_dst_v)
        pltpu.sync_copy(w_hbm.at[pl.ds(base, CHUNK)], w_v)
        pltpu.async_copy(h_hbm.at[core].at[idx_src_v], rows_v, sem).wait()

        def _scale(g, _):
            wv = w_v[pl.ds(g * LANES, LANES)]
            for e in range(LANES):
                j = g * LANES + e
                we = wv[e]
                for k in range(DH // LANES):
                    sl = pl.ds(k * LANES, LANES)
                    rows_v[j, sl] = rows_v[j, sl] * we
            return ()

        lax.fori_loop(0, CHUNK // LANES, _scale, ())
        pltpu.sync_copy(rows_v, acc_shared.at[idx_dst_v], add=True)
        return ()

    lax.fori_loop(0, N_CHUNKS, _edge_chunk, ())
    plsc.subcore_barrier()

    r0 = sub * ROWS_PER_TILE
    pltpu.sync_copy(acc_shared.at[pl.ds(r0, ROWS_PER_TILE)],
                    out_hbm.at[core, pl.ds(r0, ROWS_PER_TILE)])


def _spmm_split(h_split, src, dst, w):
    mesh = plsc.VectorSubcoreMesh(core_axis_name="c", subcore_axis_name="s",
                                  num_cores=NC, num_subcores=NS)
    return pl.kernel(
        _spmm_body,
        out_type=jax.ShapeDtypeStruct((NC, N_PAD, DH), jnp.float32),
        mesh=mesh,
        scratch_types=[
            pltpu.VMEM((CHUNK,), jnp.int32),
            pltpu.VMEM((CHUNK,), jnp.int32),
            pltpu.VMEM((CHUNK, DH), jnp.float32),
            pltpu.VMEM((ROWS_PER_TILE, DH), jnp.float32),
            pltpu.VMEM((CHUNK,), jnp.float32),
            pltpu.VMEM_SHARED((N_PAD, DH), jnp.float32),
            pltpu.SemaphoreType.DMA,
        ],
        compiler_params=pltpu.CompilerParams(use_tc_tiling_on_sc=False),
    )(h_split, src, dst, w)


ROW_BLK = 1000


def _split_body(x_ref, out_ref):
    out_ref[0] = x_ref[:, :DH]
    out_ref[1] = x_ref[:, DH:]


def _split(x):
    grid = (N_NODES // ROW_BLK,)
    return pl.pallas_call(
        _split_body,
        grid=grid,
        in_specs=[pl.BlockSpec((ROW_BLK, D), lambda i: (i, 0))],
        out_specs=pl.BlockSpec((NC, ROW_BLK, DH), lambda i: (0, i, 0)),
        out_shape=jax.ShapeDtypeStruct((NC, N_PAD, DH), jnp.float32),
    )(x)


def _fuse_matmul_body(x_ref, parts_ref, w0_ref, w1_ref, z_ref):
    h1 = jnp.concatenate([parts_ref[0], parts_ref[1]], axis=1)
    z = (jnp.dot(x_ref[...], w0_ref[...], preferred_element_type=jnp.float32)
         + jnp.dot(h1, w1_ref[...], preferred_element_type=jnp.float32))
    z_ref[0] = z[:, :DH]
    z_ref[1] = z[:, DH:]


def _fuse_matmul(x, parts, w0, w1):
    grid = (N_NODES // ROW_BLK,)
    return pl.pallas_call(
        _fuse_matmul_body,
        grid=grid,
        in_specs=[
            pl.BlockSpec((ROW_BLK, D), lambda i: (i, 0)),
            pl.BlockSpec((NC, ROW_BLK, DH), lambda i: (0, i, 0)),
            pl.BlockSpec((D, D), lambda i: (0, 0)),
            pl.BlockSpec((D, D), lambda i: (0, 0)),
        ],
        out_specs=pl.BlockSpec((NC, ROW_BLK, DH), lambda i: (0, i, 0)),
        out_shape=jax.ShapeDtypeStruct((NC, N_PAD, DH), jnp.float32),
    )(x, parts, w0, w1)


def _combine_bias_body(parts_ref, b_ref, out_ref):
    out_ref[...] = (jnp.concatenate([parts_ref[0], parts_ref[1]], axis=1)
                    + b_ref[...])


def _combine_bias(parts, b):
    grid = (N_NODES // ROW_BLK,)
    return pl.pallas_call(
        _combine_bias_body,
        grid=grid,
        in_specs=[
            pl.BlockSpec((NC, ROW_BLK, DH), lambda i: (0, i, 0)),
            pl.BlockSpec((1, D), lambda i: (0, 0)),
        ],
        out_specs=pl.BlockSpec((ROW_BLK, D), lambda i: (i, 0)),
        out_shape=jax.ShapeDtypeStruct((N_NODES, D), jnp.float32),
    )(parts, b)


def kernel(x, edge_index, edge_weight, W0, W1, b):
    dst = edge_index[0].astype(jnp.int32)
    src = edge_index[1].astype(jnp.int32)
    w = edge_weight.astype(jnp.float32)
    x_split = _split(x)
    h1_parts = _spmm_split(x_split, src, dst, w)
    z_split = _fuse_matmul(x, h1_parts, W0, W1)
    out_parts = _spmm_split(z_split, src, dst, w)
    return _combine_bias(out_parts, b.reshape(1, D))


# speedup vs baseline: 4.1490x; 2.1619x over previous
import jax
import jax.numpy as jnp
from jax import lax
from jax.experimental import pallas as pl
from jax.experimental.pallas import tpu as pltpu
from jax.experimental.pallas import tpu_sc as plsc

N_NODES = 10000
N_EDGES = 320000
D = 128
DH = D // 2
NC = 2
NS = 16
LANES = 16

EDGES_PER_TILE = N_EDGES // NS
CHUNK = 80
N_CHUNKS = EDGES_PER_TILE // CHUNK
ROWS_PER_TILE = 632
N_PAD = ROWS_PER_TILE * NS


def _spmm_body(h_hbm, src_hbm, dst_hbm, w_hbm, out_hbm,
               src_v, dst_v, w_v, rows0, rows1,
               acc_shared, sem_i, sg0, sg1, ss0, ss1):
    core = lax.axis_index("c")
    sub = lax.axis_index("s")

    pltpu.async_copy(src_hbm.at[sub], src_v, sem_i)
    pltpu.async_copy(dst_hbm.at[sub], dst_v, sem_i)
    pltpu.async_copy(w_hbm.at[sub], w_v, sem_i)

    zv = jnp.zeros((LANES,), jnp.float32)

    def _zero(j, _):
        r = j // (DH // LANES)
        k = j % (DH // LANES)
        rows0[r, pl.ds(k * LANES, LANES)] = zv
        return ()

    lax.fori_loop(0, CHUNK * (DH // LANES), _zero, (), unroll=8)
    r0 = sub * ROWS_PER_TILE
    for i in range(ROWS_PER_TILE // CHUNK):
        pltpu.sync_copy(rows0, acc_shared.at[pl.ds(r0 + i * CHUNK, CHUNK)])
    rem = ROWS_PER_TILE % CHUNK
    pltpu.sync_copy(rows0.at[pl.ds(0, rem)],
                    acc_shared.at[pl.ds(r0 + (ROWS_PER_TILE // CHUNK) * CHUNK, rem)])
    pltpu.make_async_copy(src_hbm.at[sub], src_v, sem_i).wait()
    pltpu.make_async_copy(dst_hbm.at[sub], dst_v, sem_i).wait()
    pltpu.make_async_copy(w_hbm.at[sub], w_v, sem_i).wait()
    plsc.subcore_barrier()

    def _gather(i, rows, sem):
        pltpu.async_copy(h_hbm.at[core].at[src_v.at[i]], rows, sem)

    def _scale(rows, i):
        def _grp(g, _):
            wv = w_v[i, pl.ds(g * LANES, LANES)]
            for e in range(LANES):
                j = g * LANES + e
                we = wv[e]
                for k in range(DH // LANES):
                    sl = pl.ds(k * LANES, LANES)
                    rows[j, sl] = rows[j, sl] * we
            return ()

        lax.fori_loop(0, CHUNK // LANES, _grp, ())

    def _scatter(i, rows, sem):
        pltpu.async_copy(rows, acc_shared.at[dst_v.at[i]], sem, add=True)

    def _pair(g, _):
        i0 = 2 * g
        i1 = i0 + 1

        @pl.when(g == 0)
        def _prime():
            _gather(0, rows0, sg0)

        pltpu.make_async_copy(h_hbm.at[core].at[src_v.at[i0]], rows0, sg0).wait()

        @pl.when(g > 0)
        def _free1():
            pltpu.make_async_copy(rows1, acc_shared.at[dst_v.at[i1]], ss1).wait()

        _gather(i1, rows1, sg1)
        _scale(rows0, i0)
        _scatter(i0, rows0, ss0)
        pltpu.make_async_copy(h_hbm.at[core].at[src_v.at[i1]], rows1, sg1).wait()
        pltpu.make_async_copy(rows0, acc_shared.at[dst_v.at[i0]], ss0).wait()

        @pl.when(g < N_CHUNKS // 2 - 1)
        def _next0():
            _gather(i0 + 2, rows0, sg0)

        _scale(rows1, i1)
        _scatter(i1, rows1, ss1)
        return ()

    lax.fori_loop(0, N_CHUNKS // 2, _pair, ())
    pltpu.make_async_copy(rows1, acc_shared.at[dst_v.at[N_CHUNKS - 1]], ss1).wait()
    plsc.subcore_barrier()

    pltpu.sync_copy(acc_shared.at[pl.ds(r0, ROWS_PER_TILE)],
                    out_hbm.at[core, pl.ds(r0, ROWS_PER_TILE)])


def _spmm_split(h_split, src, dst, w):
    mesh = plsc.VectorSubcoreMesh(core_axis_name="c", subcore_axis_name="s",
                                  num_cores=NC, num_subcores=NS)
    return pl.kernel(
        _spmm_body,
        out_type=jax.ShapeDtypeStruct((NC, N_PAD, DH), jnp.float32),
        mesh=mesh,
        scratch_types=[
            pltpu.VMEM((N_CHUNKS, CHUNK), jnp.int32),
            pltpu.VMEM((N_CHUNKS, CHUNK), jnp.int32),
            pltpu.VMEM((N_CHUNKS, CHUNK), jnp.float32),
            pltpu.VMEM((CHUNK, DH), jnp.float32),
            pltpu.VMEM((CHUNK, DH), jnp.float32),
            pltpu.VMEM_SHARED((N_PAD, DH), jnp.float32),
            pltpu.SemaphoreType.DMA,
            pltpu.SemaphoreType.DMA,
            pltpu.SemaphoreType.DMA,
            pltpu.SemaphoreType.DMA,
            pltpu.SemaphoreType.DMA,
        ],
        compiler_params=pltpu.CompilerParams(use_tc_tiling_on_sc=False),
    )(h_split, src, dst, w)


ROW_BLK = 1000


def _split_body(x_ref, out_ref):
    out_ref[0] = x_ref[:, :DH]
    out_ref[1] = x_ref[:, DH:]


def _split(x):
    grid = (N_NODES // ROW_BLK,)
    return pl.pallas_call(
        _split_body,
        grid=grid,
        in_specs=[pl.BlockSpec((ROW_BLK, D), lambda i: (i, 0))],
        out_specs=pl.BlockSpec((NC, ROW_BLK, DH), lambda i: (0, i, 0)),
        out_shape=jax.ShapeDtypeStruct((NC, N_PAD, DH), jnp.float32),
    )(x)


def _fuse_matmul_body(x_ref, parts_ref, w0_ref, w1_ref, z_ref):
    h1 = jnp.concatenate([parts_ref[0], parts_ref[1]], axis=1)
    z = (jnp.dot(x_ref[...], w0_ref[...], preferred_element_type=jnp.float32)
         + jnp.dot(h1, w1_ref[...], preferred_element_type=jnp.float32))
    z_ref[0] = z[:, :DH]
    z_ref[1] = z[:, DH:]


def _fuse_matmul(x, parts, w0, w1):
    grid = (N_NODES // ROW_BLK,)
    return pl.pallas_call(
        _fuse_matmul_body,
        grid=grid,
        in_specs=[
            pl.BlockSpec((ROW_BLK, D), lambda i: (i, 0)),
            pl.BlockSpec((NC, ROW_BLK, DH), lambda i: (0, i, 0)),
            pl.BlockSpec((D, D), lambda i: (0, 0)),
            pl.BlockSpec((D, D), lambda i: (0, 0)),
        ],
        out_specs=pl.BlockSpec((NC, ROW_BLK, DH), lambda i: (0, i, 0)),
        out_shape=jax.ShapeDtypeStruct((NC, N_PAD, DH), jnp.float32),
    )(x, parts, w0, w1)


def _combine_bias_body(parts_ref, b_ref, out_ref):
    out_ref[...] = (jnp.concatenate([parts_ref[0], parts_ref[1]], axis=1)
                    + b_ref[...])


def _combine_bias(parts, b):
    grid = (N_NODES // ROW_BLK,)
    return pl.pallas_call(
        _combine_bias_body,
        grid=grid,
        in_specs=[
            pl.BlockSpec((NC, ROW_BLK, DH), lambda i: (0, i, 0)),
            pl.BlockSpec((1, D), lambda i: (0, 0)),
        ],
        out_specs=pl.BlockSpec((ROW_BLK, D), lambda i: (i, 0)),
        out_shape=jax.ShapeDtypeStruct((N_NODES, D), jnp.float32),
    )(parts, b)


def kernel(x, edge_index, edge_weight, W0, W1, b):
    dst = edge_index[0].astype(jnp.int32).reshape(NS, N_CHUNKS, CHUNK)
    src = edge_index[1].astype(jnp.int32).reshape(NS, N_CHUNKS, CHUNK)
    w = edge_weight.astype(jnp.float32).reshape(NS, N_CHUNKS, CHUNK)
    x_split = _split(x)
    h1_parts = _spmm_split(x_split, src, dst, w)
    z_split = _fuse_matmul(x, h1_parts, W0, W1)
    out_parts = _spmm_split(z_split, src, dst, w)
    return _combine_bias(out_parts, b.reshape(1, D))
